# baseline (speedup 1.0000x reference)
import functools
import math

import jax
import jax.numpy as jnp
from jax import lax
from jax.experimental import pallas as pl
from jax.experimental import layout as jex_layout
from jax.experimental.pallas import tpu as pltpu
from jax.experimental.pallas import tpu_sc as plsc

D = 64
SCALE = math.sqrt(D)
NC, NS = 2, 16
NW = NC * NS
IB = 128
K = 1


@functools.lru_cache(maxsize=None)
def _emb_kernel(B):
    rows_per_w = B // (NW * IB)
    ngroups = rows_per_w // K
    assert ngroups % 2 == 0
    npairs = ngroups // 2
    mesh = plsc.VectorSubcoreMesh(
        core_axis_name="c", subcore_axis_name="s",
        num_cores=NC, num_subcores=NS)

    @functools.partial(
        pl.kernel,
        out_type=jax.ShapeDtypeStruct((B, D), jnp.float32),
        mesh=mesh,
        scratch_types=[
            pltpu.VMEM((rows_per_w, IB), jnp.int32),
            pltpu.VMEM((K * IB, 2 * D), jnp.float32),
            pltpu.VMEM((K * IB, 2 * D), jnp.float32),
            pltpu.VMEM((K * IB, D), jnp.float32),
            pltpu.VMEM((K * IB, D), jnp.float32),
            pltpu.SemaphoreType.DMA,
            pltpu.SemaphoreType.DMA,
            pltpu.SemaphoreType.DMA,
            pltpu.SemaphoreType.DMA,
        ],
        compiler_params=pltpu.CompilerParams(use_tc_tiling_on_sc=True),
    )
    def body(x_hbm, lut_hbm, out_hbm, idx_v, rows_a, rows_b, cmp_a, cmp_b,
             gsem_a, gsem_b, ssem_a, ssem_b):
        wid = lax.axis_index("s") * NC + lax.axis_index("c")
        base_row = wid * rows_per_w
        out_base = base_row * IB
        pltpu.sync_copy(x_hbm.at[pl.ds(base_row, rows_per_w)], idx_v)

        def start_gather(g, rows_v, gsem):
            pltpu.async_copy(lut_hbm.at[idx_v.at[g]], rows_v, gsem)

        def drain_gather(rows_v, gsem):
            pltpu.make_async_copy(lut_hbm.at[idx_v.at[0]], rows_v,
                                  gsem).wait()

        def scale(rows_v, cmp_v):
            def row(r, c):
                for k in range(D // 16):
                    sl = pl.ds(k * 16, 16)
                    cmp_v[r, sl] = rows_v[r, sl]
                return c
            lax.fori_loop(0, IB, row, 0)

        def start_scatter(g, cmp_v, ssem):
            pltpu.async_copy(
                cmp_v, out_hbm.at[pl.ds(out_base + g * IB, IB)], ssem)

        def drain_scatter(g, cmp_v, ssem):
            pltpu.make_async_copy(
                cmp_v, out_hbm.at[pl.ds(out_base + g * IB, IB)], ssem).wait()

        start_gather(0, rows_a, gsem_a)
        start_gather(1, rows_b, gsem_b)

        def pair(i2, c):
            ga = 2 * i2
            drain_gather(rows_a, gsem_a)
            scale(rows_a, cmp_a)
            start_scatter(ga, cmp_a, ssem_a)
            start_gather(ga + 2, rows_a, gsem_a)
            drain_gather(rows_b, gsem_b)
            scale(rows_b, cmp_b)
            start_scatter(ga + 1, cmp_b, ssem_b)
            start_gather(ga + 3, rows_b, gsem_b)
            drain_scatter(ga, cmp_a, ssem_a)
            drain_scatter(ga + 1, cmp_b, ssem_b)
            return c

        lax.fori_loop(0, npairs - 1, pair, 0)

        gl = ngroups - 2
        drain_gather(rows_a, gsem_a)
        scale(rows_a, cmp_a)
        start_scatter(gl, cmp_a, ssem_a)
        drain_gather(rows_b, gsem_b)
        scale(rows_b, cmp_b)
        start_scatter(gl + 1, cmp_b, ssem_b)
        drain_scatter(gl, cmp_a, ssem_a)
        drain_scatter(gl + 1, cmp_b, ssem_b)

    return body


@functools.lru_cache(maxsize=None)
def _jitted():
    try:
        dev = jax.devices("tpu")[0]
        fmt = jex_layout.Format(
            jex_layout.Layout(major_to_minor=(0, 1, 2)),
            jax.sharding.SingleDeviceSharding(dev))
        return jax.jit(_impl, out_shardings=fmt)
    except Exception:
        return jax.jit(_impl)


def kernel(x, lut):
    return _jitted()(x, lut)


TB = 512


def _widen_body(x_ref, o_ref):
    t = jnp.swapaxes(x_ref[...], 0, 1) * SCALE
    o_ref[...] = jnp.concatenate([t, t], axis=1)


@functools.lru_cache(maxsize=None)
def _lut_widen(v):
    grid = (v + TB - 1) // TB
    return pl.pallas_call(
        _widen_body,
        grid=(grid,),
        in_specs=[pl.BlockSpec((D, TB), lambda i: (0, i))],
        out_specs=pl.BlockSpec((TB, 2 * D), lambda i: (i, 0)),
        out_shape=jax.ShapeDtypeStruct((v, 2 * D), jnp.float32),
    )


def _impl(x, lut):
    s0, s1 = x.shape
    B = s0 * s1
    x2 = x.reshape(B // IB, IB).astype(jnp.int32)
    lut_w = _lut_widen(lut.shape[0])(jnp.swapaxes(lut, 0, 1))
    out = _emb_kernel(B)(x2, lut_w)
    return out.reshape(s0, s1, D)


# speedup vs baseline: 2.1646x; 2.1646x over previous
import functools
import math

import jax
import jax.numpy as jnp
from jax import lax
from jax.experimental import pallas as pl
from jax.experimental import layout as jex_layout
from jax.experimental.pallas import tpu as pltpu
from jax.experimental.pallas import tpu_sc as plsc

D = 64
SCALE = math.sqrt(D)
NC, NS = 2, 16
NW = NC * NS
IB = 128
K = 1


@functools.lru_cache(maxsize=None)
def _emb_kernel(B):
    rows_per_w = B // (NW * IB)
    ngroups = rows_per_w // K
    assert ngroups % 2 == 0
    npairs = ngroups // 2
    mesh = plsc.VectorSubcoreMesh(
        core_axis_name="c", subcore_axis_name="s",
        num_cores=NC, num_subcores=NS)

    @functools.partial(
        pl.kernel,
        out_type=jax.ShapeDtypeStruct((B, D), jnp.float32),
        mesh=mesh,
        scratch_types=[
            pltpu.VMEM((rows_per_w, IB), jnp.int32),
            pltpu.VMEM((K * IB, 2 * D), jnp.float32),
            pltpu.VMEM((K * IB, 2 * D), jnp.float32),
            pltpu.VMEM((K * IB, D), jnp.float32),
            pltpu.VMEM((K * IB, D), jnp.float32),
            pltpu.SemaphoreType.DMA,
            pltpu.SemaphoreType.DMA,
            pltpu.SemaphoreType.DMA,
            pltpu.SemaphoreType.DMA,
        ],
        compiler_params=pltpu.CompilerParams(use_tc_tiling_on_sc=True),
    )
    def body(x_hbm, lut_hbm, out_hbm, idx_v, rows_a, rows_b, cmp_a, cmp_b,
             gsem_a, gsem_b, ssem_a, ssem_b):
        wid = lax.axis_index("s") * NC + lax.axis_index("c")
        base_row = wid * rows_per_w
        out_base = base_row * IB
        pltpu.sync_copy(x_hbm.at[pl.ds(base_row, rows_per_w)], idx_v)

        def start_gather(g, rows_v, gsem):
            pltpu.async_copy(lut_hbm.at[idx_v.at[g]], rows_v, gsem)

        def drain_gather(rows_v, gsem):
            pltpu.make_async_copy(lut_hbm.at[idx_v.at[0]], rows_v,
                                  gsem).wait()

        def scale(rows_v, cmp_v):
            def row(r, c):
                for k in range(D // 16):
                    sl = pl.ds(k * 16, 16)
                    cmp_v[r, sl] = rows_v[r, sl]
                return c
            lax.fori_loop(0, IB, row, 0)

        def start_scatter(g, cmp_v, ssem):
            pltpu.async_copy(
                cmp_v, out_hbm.at[pl.ds(out_base + g * IB, IB)], ssem)

        def drain_scatter(g, cmp_v, ssem):
            pltpu.make_async_copy(
                cmp_v, out_hbm.at[pl.ds(out_base + g * IB, IB)], ssem).wait()

        start_gather(0, rows_a, gsem_a)
        start_gather(1, rows_b, gsem_b)

        def pair(i2, c):
            ga = 2 * i2
            drain_gather(rows_a, gsem_a)
            scale(rows_a, cmp_a)
            start_scatter(ga, cmp_a, ssem_a)
            start_gather(ga + 2, rows_a, gsem_a)
            drain_gather(rows_b, gsem_b)
            scale(rows_b, cmp_b)
            start_scatter(ga + 1, cmp_b, ssem_b)
            start_gather(ga + 3, rows_b, gsem_b)
            drain_scatter(ga, cmp_a, ssem_a)
            drain_scatter(ga + 1, cmp_b, ssem_b)
            return c

        lax.fori_loop(0, npairs - 1, pair, 0)

        gl = ngroups - 2
        drain_gather(rows_a, gsem_a)
        scale(rows_a, cmp_a)
        start_scatter(gl, cmp_a, ssem_a)
        drain_gather(rows_b, gsem_b)
        scale(rows_b, cmp_b)
        start_scatter(gl + 1, cmp_b, ssem_b)
        drain_scatter(gl, cmp_a, ssem_a)
        drain_scatter(gl + 1, cmp_b, ssem_b)

    return body


@functools.lru_cache(maxsize=None)
def _jitted():
    try:
        dev = jax.devices("tpu")[0]
        fmt = jex_layout.Format(
            jex_layout.Layout(major_to_minor=(0, 1, 2)),
            jax.sharding.SingleDeviceSharding(dev))
        return jax.jit(_impl, out_shardings=fmt)
    except Exception:
        return jax.jit(_impl)


def kernel(x, lut):
    return _jitted()(x, lut)


TB = 8192


def _widen_body(x_ref, o_ref):
    t = jnp.swapaxes(x_ref[...], 0, 1) * SCALE
    o_ref[...] = jnp.concatenate([t, t], axis=1)


@functools.lru_cache(maxsize=None)
def _lut_widen(v):
    grid = (v + TB - 1) // TB
    return pl.pallas_call(
        _widen_body,
        grid=(grid,),
        in_specs=[pl.BlockSpec((D, TB), lambda i: (0, i))],
        out_specs=pl.BlockSpec((TB, 2 * D), lambda i: (i, 0)),
        out_shape=jax.ShapeDtypeStruct((v, 2 * D), jnp.float32),
    )


def _impl(x, lut):
    s0, s1 = x.shape
    B = s0 * s1
    x2 = x.reshape(B // IB, IB).astype(jnp.int32)
    lut_w = _lut_widen(lut.shape[0])(jnp.swapaxes(lut, 0, 1))
    out = _emb_kernel(B)(x2, lut_w)
    return out.reshape(s0, s1, D)


# speedup vs baseline: 2.2586x; 1.0434x over previous
import functools
import math

import jax
import jax.numpy as jnp
from jax import lax
from jax.experimental import pallas as pl
from jax.experimental import layout as jex_layout
from jax.experimental.pallas import tpu as pltpu
from jax.experimental.pallas import tpu_sc as plsc

D = 64
SCALE = math.sqrt(D)
NC, NS = 2, 16
NW = NC * NS
IB = 128
K = 1


@functools.lru_cache(maxsize=None)
def _emb_kernel(B):
    rows_per_w = B // (NW * IB)
    ngroups = rows_per_w // K
    assert ngroups % 2 == 0
    npairs = ngroups // 2
    mesh = plsc.VectorSubcoreMesh(
        core_axis_name="c", subcore_axis_name="s",
        num_cores=NC, num_subcores=NS)

    @functools.partial(
        pl.kernel,
        out_type=jax.ShapeDtypeStruct((B, D), jnp.float32),
        mesh=mesh,
        scratch_types=[
            pltpu.VMEM((rows_per_w, IB), jnp.int32),
            pltpu.VMEM((K * IB, 2 * D), jnp.float32),
            pltpu.VMEM((K * IB, 2 * D), jnp.float32),
            pltpu.VMEM((K * IB, D), jnp.float32),
            pltpu.VMEM((K * IB, D), jnp.float32),
            pltpu.SemaphoreType.DMA,
            pltpu.SemaphoreType.DMA,
            pltpu.SemaphoreType.DMA,
            pltpu.SemaphoreType.DMA,
        ],
        compiler_params=pltpu.CompilerParams(use_tc_tiling_on_sc=True),
    )
    def body(x_hbm, lut_hbm, out_hbm, idx_v, rows_a, rows_b, cmp_a, cmp_b,
             gsem_a, gsem_b, ssem_a, ssem_b):
        wid = lax.axis_index("s") * NC + lax.axis_index("c")
        base_row = wid * rows_per_w
        out_base = base_row * IB
        pltpu.sync_copy(x_hbm.at[pl.ds(base_row, rows_per_w)], idx_v)

        def start_gather(g, rows_v, gsem):
            pltpu.async_copy(lut_hbm.at[idx_v.at[g]], rows_v, gsem)

        def drain_gather(rows_v, gsem):
            pltpu.make_async_copy(lut_hbm.at[idx_v.at[0]], rows_v,
                                  gsem).wait()

        def scale(rows_v, cmp_v):
            def row(r, c):
                for k in range(D // 16):
                    sl = pl.ds(k * 16, 16)
                    cmp_v[r, sl] = rows_v[r, sl]
                return c
            lax.fori_loop(0, IB, row, 0)

        def start_scatter(g, cmp_v, ssem):
            pltpu.async_copy(
                cmp_v, out_hbm.at[pl.ds(out_base + g * IB, IB)], ssem)

        def drain_scatter(g, cmp_v, ssem):
            pltpu.make_async_copy(
                cmp_v, out_hbm.at[pl.ds(out_base + g * IB, IB)], ssem).wait()

        start_gather(0, rows_a, gsem_a)
        start_gather(1, rows_b, gsem_b)

        def pair(i2, c):
            ga = 2 * i2
            drain_gather(rows_a, gsem_a)
            scale(rows_a, cmp_a)
            start_scatter(ga, cmp_a, ssem_a)
            start_gather(ga + 2, rows_a, gsem_a)
            drain_gather(rows_b, gsem_b)
            scale(rows_b, cmp_b)
            start_scatter(ga + 1, cmp_b, ssem_b)
            start_gather(ga + 3, rows_b, gsem_b)
            drain_scatter(ga, cmp_a, ssem_a)
            drain_scatter(ga + 1, cmp_b, ssem_b)
            return c

        lax.fori_loop(0, npairs - 1, pair, 0)

        gl = ngroups - 2
        drain_gather(rows_a, gsem_a)
        scale(rows_a, cmp_a)
        start_scatter(gl, cmp_a, ssem_a)
        drain_gather(rows_b, gsem_b)
        scale(rows_b, cmp_b)
        start_scatter(gl + 1, cmp_b, ssem_b)
        drain_scatter(gl, cmp_a, ssem_a)
        drain_scatter(gl + 1, cmp_b, ssem_b)

    return body


@functools.lru_cache(maxsize=None)
def _jitted():
    try:
        dev = jax.devices("tpu")[0]
        fmt = jex_layout.Format(
            jex_layout.Layout(major_to_minor=(0, 1, 2)),
            jax.sharding.SingleDeviceSharding(dev))
        return jax.jit(_impl, out_shardings=fmt)
    except Exception:
        return jax.jit(_impl)


def kernel(x, lut):
    return _jitted()(x, lut)


TB = 16384


def _widen_body(x_ref, o_ref):
    t = jnp.swapaxes(x_ref[...], 0, 1) * SCALE
    o_ref[...] = jnp.concatenate([t, t], axis=1)


@functools.lru_cache(maxsize=None)
def _lut_widen(v):
    grid = (v + TB - 1) // TB
    return pl.pallas_call(
        _widen_body,
        grid=(grid,),
        in_specs=[pl.BlockSpec((D, TB), lambda i: (0, i))],
        out_specs=pl.BlockSpec((TB, 2 * D), lambda i: (i, 0)),
        out_shape=jax.ShapeDtypeStruct((v, 2 * D), jnp.float32),
    )


def _impl(x, lut):
    s0, s1 = x.shape
    B = s0 * s1
    x2 = x.reshape(B // IB, IB).astype(jnp.int32)
    lut_w = _lut_widen(lut.shape[0])(jnp.swapaxes(lut, 0, 1))
    out = _emb_kernel(B)(x2, lut_w)
    return out.reshape(s0, s1, D)


# speedup vs baseline: 2.2613x; 1.0012x over previous
import functools
import math

import jax
import jax.numpy as jnp
from jax import lax
from jax.experimental import pallas as pl
from jax.experimental.pallas import tpu as pltpu
from jax.experimental.pallas import tpu_sc as plsc

D = 64
SCALE = math.sqrt(D)
NC, NS = 2, 16
NW = NC * NS
IB = 128
K = 1


@functools.lru_cache(maxsize=None)
def _emb_kernel(B):
    rows_per_w = B // (NW * IB)
    ngroups = rows_per_w // K
    assert ngroups % 2 == 0
    npairs = ngroups // 2
    mesh = plsc.VectorSubcoreMesh(
        core_axis_name="c", subcore_axis_name="s",
        num_cores=NC, num_subcores=NS)

    @functools.partial(
        pl.kernel,
        out_type=jax.ShapeDtypeStruct((B, D), jnp.float32),
        mesh=mesh,
        scratch_types=[
            pltpu.VMEM((rows_per_w, IB), jnp.int32),
            pltpu.VMEM((K * IB, 2 * D), jnp.float32),
            pltpu.VMEM((K * IB, 2 * D), jnp.float32),
            pltpu.VMEM((K * IB, D), jnp.float32),
            pltpu.VMEM((K * IB, D), jnp.float32),
            pltpu.SemaphoreType.DMA,
            pltpu.SemaphoreType.DMA,
            pltpu.SemaphoreType.DMA,
            pltpu.SemaphoreType.DMA,
        ],
        compiler_params=pltpu.CompilerParams(use_tc_tiling_on_sc=True),
    )
    def body(x_hbm, lut_hbm, out_hbm, idx_v, rows_a, rows_b, cmp_a, cmp_b,
             gsem_a, gsem_b, ssem_a, ssem_b):
        wid = lax.axis_index("s") * NC + lax.axis_index("c")
        base_row = wid * rows_per_w
        out_base = base_row * IB
        pltpu.sync_copy(x_hbm.at[pl.ds(base_row, rows_per_w)], idx_v)

        def start_gather(g, rows_v, gsem):
            pltpu.async_copy(lut_hbm.at[idx_v.at[g]], rows_v, gsem)

        def drain_gather(rows_v, gsem):
            pltpu.make_async_copy(lut_hbm.at[idx_v.at[0]], rows_v,
                                  gsem).wait()

        def scale(rows_v, cmp_v):
            def row(r, c):
                for k in range(D // 16):
                    sl = pl.ds(k * 16, 16)
                    cmp_v[r, sl] = rows_v[r, sl]
                return c
            lax.fori_loop(0, IB, row, 0)

        def start_scatter(g, cmp_v, ssem):
            pltpu.async_copy(
                cmp_v, out_hbm.at[pl.ds(out_base + g * IB, IB)], ssem)

        def drain_scatter(g, cmp_v, ssem):
            pltpu.make_async_copy(
                cmp_v, out_hbm.at[pl.ds(out_base + g * IB, IB)], ssem).wait()

        start_gather(0, rows_a, gsem_a)
        start_gather(1, rows_b, gsem_b)

        def pair(i2, c):
            ga = 2 * i2
            drain_gather(rows_a, gsem_a)
            scale(rows_a, cmp_a)
            start_scatter(ga, cmp_a, ssem_a)
            start_gather(ga + 2, rows_a, gsem_a)
            drain_gather(rows_b, gsem_b)
            scale(rows_b, cmp_b)
            start_scatter(ga + 1, cmp_b, ssem_b)
            start_gather(ga + 3, rows_b, gsem_b)
            drain_scatter(ga, cmp_a, ssem_a)
            drain_scatter(ga + 1, cmp_b, ssem_b)
            return c

        lax.fori_loop(0, npairs - 1, pair, 0)

        gl = ngroups - 2
        drain_gather(rows_a, gsem_a)
        scale(rows_a, cmp_a)
        start_scatter(gl, cmp_a, ssem_a)
        drain_gather(rows_b, gsem_b)
        scale(rows_b, cmp_b)
        start_scatter(gl + 1, cmp_b, ssem_b)
        drain_scatter(gl, cmp_a, ssem_a)
        drain_scatter(gl + 1, cmp_b, ssem_b)

    return body


TB = 16384


def _widen_body(x_ref, o_ref):
    t = jnp.swapaxes(x_ref[...], 0, 1) * SCALE
    o_ref[...] = jnp.concatenate([t, t], axis=1)


@functools.lru_cache(maxsize=None)
def _lut_widen(v):
    grid = (v + TB - 1) // TB
    return pl.pallas_call(
        _widen_body,
        grid=(grid,),
        in_specs=[pl.BlockSpec((D, TB), lambda i: (0, i))],
        out_specs=pl.BlockSpec((TB, 2 * D), lambda i: (i, 0)),
        out_shape=jax.ShapeDtypeStruct((v, 2 * D), jnp.float32),
    )


@jax.jit
def kernel(x, lut):
    s0, s1 = x.shape
    B = s0 * s1
    x2 = x.reshape(B // IB, IB).astype(jnp.int32)
    lut_w = _lut_widen(lut.shape[0])(jnp.swapaxes(lut, 0, 1))
    out = _emb_kernel(B)(x2, lut_w)
    return out.reshape(s0, s1, D)
